# Initial kernel scaffold; baseline (speedup 1.0000x reference)
import jax
import jax.numpy as jnp
from jax.experimental import pallas as pl


def kernel(x, x_reactant, edge_index, edge_attr, u, u_reactant, batch):
    raise NotImplementedError("write your pallas kernel here")


# speedup vs baseline: 65.1358x; 65.1358x over previous
import functools

import jax
import jax.numpy as jnp
from jax import lax
from jax.experimental import pallas as pl
from jax.experimental.pallas import tpu as pltpu
from jax.experimental.pallas import tpu_sc as plsc

_N_NODES = 10000
_D = 128
_G = 100
_CHUNK = 80
_NCHUNKS = _N_NODES // _CHUNK
_NW = 32
_KMAX = -(-_NCHUNKS // _NW)


def _sc_segment_sums(x, x_reactant, batch_i32, z128, ones_chunk):
    mesh = plsc.VectorSubcoreMesh(core_axis_name="c", subcore_axis_name="s")

    @functools.partial(
        pl.kernel,
        mesh=mesh,
        out_type=(
            jax.ShapeDtypeStruct((2, _G, _D), jnp.float32),
            jax.ShapeDtypeStruct((2, _G, _D), jnp.float32),
            jax.ShapeDtypeStruct((2, _G, _D), jnp.float32),
        ),
        scratch_types=[
            pltpu.VMEM((_CHUNK, _D), jnp.float32),
            pltpu.VMEM((_CHUNK, _D), jnp.float32),
            pltpu.VMEM((_CHUNK,), jnp.int32),
            pltpu.VMEM((_CHUNK, _D), jnp.float32),
            pltpu.VMEM_SHARED((_G, _D), jnp.float32),
            pltpu.VMEM_SHARED((_G, _D), jnp.float32),
            pltpu.VMEM_SHARED((_G, _D), jnp.float32),
        ],
    )
    def seg_sum_kernel(x_hbm, xr_hbm, b_hbm, z128_hbm, ones_hbm,
                       outx_hbm, outxr_hbm, outc_hbm,
                       xa_v, xb_v, idx_v, ones_v, accx_s, accxr_s, accc_s):
        c = lax.axis_index("c")
        s = lax.axis_index("s")
        wid = s * 2 + c

        @pl.when(s == 0)
        def _zero_shared():
            pltpu.sync_copy(z128_hbm, accx_s)
            pltpu.sync_copy(z128_hbm, accxr_s)
            pltpu.sync_copy(z128_hbm, accc_s)

        pltpu.sync_copy(ones_hbm, ones_v)
        plsc.subcore_barrier()

        for k in range(_KMAX):
            j = wid + _NW * k

            @pl.when(j < _NCHUNKS)
            def _do_chunk(j=j):
                base = pl.multiple_of(j * _CHUNK, 8)
                pltpu.sync_copy(x_hbm.at[pl.ds(base, _CHUNK), :], xa_v)
                pltpu.sync_copy(xr_hbm.at[pl.ds(base, _CHUNK), :], xb_v)
                pltpu.sync_copy(b_hbm.at[pl.ds(base, _CHUNK)], idx_v)
                pltpu.sync_copy(xa_v, accx_s.at[idx_v], add=True)
                pltpu.sync_copy(xb_v, accxr_s.at[idx_v], add=True)
                pltpu.sync_copy(ones_v, accc_s.at[idx_v], add=True)

        plsc.subcore_barrier()

        @pl.when(s == 0)
        def _write_out():
            pltpu.sync_copy(accx_s, outx_hbm.at[c])
            pltpu.sync_copy(accxr_s, outxr_hbm.at[c])
            pltpu.sync_copy(accc_s, outc_hbm.at[c])

    return seg_sum_kernel(x, x_reactant, batch_i32, z128, ones_chunk)


def _finalize(u, u_reactant, accx, accxr, cnt):
    def body(u_ref, ur_ref, ax_ref, axr_ref, c_ref, o_ref):
        ssum = (ax_ref[0] - axr_ref[0]) + (ax_ref[1] - axr_ref[1])
        counts = c_ref[0] + c_ref[1]
        denom = jnp.maximum(jnp.max(counts, axis=1, keepdims=True), 1.0)
        du = u_ref[...] - ur_ref[...]
        o_ref[...] = jnp.concatenate(
            [du, ssum / denom, jnp.zeros((_G, 16), jnp.float32)], axis=1)

    return pl.pallas_call(
        body,
        out_shape=jax.ShapeDtypeStruct((_G, 32 + _D + 16), jnp.float32),
    )(u, u_reactant, accx, accxr, cnt)


def kernel(x, x_reactant, edge_index, edge_attr, u, u_reactant, batch):
    del edge_index, edge_attr
    batch_i32 = batch.astype(jnp.int32)
    z128 = jnp.zeros((_G, _D), jnp.float32)
    ones_chunk = jnp.ones((_CHUNK, _D), jnp.float32)
    accx, accxr, cnt = _sc_segment_sums(
        x, x_reactant, batch_i32, z128, ones_chunk)
    return _finalize(u, u_reactant, accx, accxr, cnt)


# speedup vs baseline: 70.7680x; 1.0865x over previous
import functools

import jax
import jax.numpy as jnp
from jax import lax
from jax.experimental import pallas as pl
from jax.experimental.pallas import tpu as pltpu
from jax.experimental.pallas import tpu_sc as plsc

_N_NODES = 10000
_D = 128
_G = 100
_CHUNK = 104
_NCHW = 3
_PER_W = _CHUNK * _NCHW
_NW = 32
_TAIL = _N_NODES - _PER_W * _NW
_TAIL_BASE = _PER_W * _NW


def _sc_segment_sums(x, x_reactant, batch_i32, z128, ones_chunk):
    mesh = plsc.VectorSubcoreMesh(core_axis_name="c", subcore_axis_name="s")

    @functools.partial(
        pl.kernel,
        mesh=mesh,
        out_type=(
            jax.ShapeDtypeStruct((2, _G, _D), jnp.float32),
            jax.ShapeDtypeStruct((2, _G, _D), jnp.float32),
            jax.ShapeDtypeStruct((2, _G, _D), jnp.float32),
        ),
        scratch_types=[
            pltpu.VMEM((_CHUNK, _D), jnp.float32),
            pltpu.VMEM((_CHUNK, _D), jnp.float32),
            pltpu.VMEM((_CHUNK, _D), jnp.float32),
            pltpu.VMEM((_CHUNK, _D), jnp.float32),
            pltpu.VMEM((_CHUNK, _D), jnp.float32),
            pltpu.VMEM((_CHUNK, _D), jnp.float32),
            pltpu.VMEM((_CHUNK,), jnp.int32),
            pltpu.VMEM((_CHUNK,), jnp.int32),
            pltpu.VMEM((_CHUNK,), jnp.int32),
            pltpu.VMEM((_CHUNK, _D), jnp.float32),
            pltpu.VMEM((_TAIL, _D), jnp.float32),
            pltpu.VMEM((_TAIL, _D), jnp.float32),
            pltpu.VMEM((_TAIL,), jnp.int32),
            pltpu.VMEM((_TAIL, _D), jnp.float32),
            pltpu.VMEM_SHARED((_G, _D), jnp.float32),
            pltpu.VMEM_SHARED((_G, _D), jnp.float32),
            pltpu.VMEM_SHARED((_G, _D), jnp.float32),
            pltpu.SemaphoreType.DMA,
            pltpu.SemaphoreType.DMA,
            pltpu.SemaphoreType.DMA,
            pltpu.SemaphoreType.DMA,
        ],
    )
    def seg_sum_kernel(x_hbm, xr_hbm, b_hbm, z128_hbm, ones_hbm,
                       outx_hbm, outxr_hbm, outc_hbm,
                       xa0, xa1, xa2, xb0, xb1, xb2, i0, i1, i2, ones_v,
                       xt, xrt, it, ones_t,
                       accx_s, accxr_s, accc_s,
                       sg0, sg1, sg2, ss):
        c = lax.axis_index("c")
        s = lax.axis_index("s")
        wid = s * 2 + c
        xa = (xa0, xa1, xa2)
        xb = (xb0, xb1, xb2)
        ii = (i0, i1, i2)
        sg = (sg0, sg1, sg2)

        @pl.when(s == 0)
        def _zero_shared():
            pltpu.sync_copy(z128_hbm, accx_s)
            pltpu.sync_copy(z128_hbm, accxr_s)
            pltpu.sync_copy(z128_hbm, accc_s)

        pltpu.sync_copy(ones_hbm, ones_v)

        base0 = wid * _PER_W
        gathers = []
        for t in range(_NCHW):
            b = pl.multiple_of(base0 + t * _CHUNK, 8)
            gathers.append((
                pltpu.async_copy(x_hbm.at[pl.ds(b, _CHUNK), :], xa[t], sg[t]),
                pltpu.async_copy(xr_hbm.at[pl.ds(b, _CHUNK), :], xb[t], sg[t]),
                pltpu.async_copy(b_hbm.at[pl.ds(b, _CHUNK)], ii[t], sg[t]),
            ))

        plsc.subcore_barrier()

        scatters = []
        for t in range(_NCHW):
            for d in gathers[t]:
                d.wait()
            scatters.append(
                pltpu.async_copy(xa[t], accx_s.at[ii[t]], ss, add=True))
            scatters.append(
                pltpu.async_copy(xb[t], accxr_s.at[ii[t]], ss, add=True))
            scatters.append(
                pltpu.async_copy(ones_v, accc_s.at[ii[t]], ss, add=True))
        for d in scatters:
            d.wait()

        @pl.when(wid == _NW - 1)
        def _tail():
            pltpu.sync_copy(x_hbm.at[pl.ds(_TAIL_BASE, _TAIL), :], xt)
            pltpu.sync_copy(xr_hbm.at[pl.ds(_TAIL_BASE, _TAIL), :], xrt)
            pltpu.sync_copy(b_hbm.at[pl.ds(_TAIL_BASE, _TAIL)], it)
            pltpu.sync_copy(ones_hbm.at[pl.ds(0, _TAIL), :], ones_t)
            pltpu.sync_copy(xt, accx_s.at[it], add=True)
            pltpu.sync_copy(xrt, accxr_s.at[it], add=True)
            pltpu.sync_copy(ones_t, accc_s.at[it], add=True)

        plsc.subcore_barrier()

        @pl.when(s == 0)
        def _write_out():
            pltpu.sync_copy(accx_s, outx_hbm.at[c])
            pltpu.sync_copy(accxr_s, outxr_hbm.at[c])
            pltpu.sync_copy(accc_s, outc_hbm.at[c])

    return seg_sum_kernel(x, x_reactant, batch_i32, z128, ones_chunk)


def _finalize(u, u_reactant, accx, accxr, cnt):
    def body(u_ref, ur_ref, ax_ref, axr_ref, c_ref, o_ref):
        ssum = (ax_ref[0] - axr_ref[0]) + (ax_ref[1] - axr_ref[1])
        counts = c_ref[0] + c_ref[1]
        denom = jnp.maximum(jnp.max(counts, axis=1, keepdims=True), 1.0)
        du = u_ref[...] - ur_ref[...]
        o_ref[...] = jnp.concatenate(
            [du, ssum / denom, jnp.zeros((_G, 16), jnp.float32)], axis=1)

    return pl.pallas_call(
        body,
        out_shape=jax.ShapeDtypeStruct((_G, 32 + _D + 16), jnp.float32),
    )(u, u_reactant, accx, accxr, cnt)


def kernel(x, x_reactant, edge_index, edge_attr, u, u_reactant, batch):
    del edge_index, edge_attr
    batch_i32 = batch.astype(jnp.int32)
    z128 = jnp.zeros((_G, _D), jnp.float32)
    ones_chunk = jnp.ones((_CHUNK, _D), jnp.float32)
    accx, accxr, cnt = _sc_segment_sums(
        x, x_reactant, batch_i32, z128, ones_chunk)
    return _finalize(u, u_reactant, accx, accxr, cnt)


# speedup vs baseline: 83.6427x; 1.1819x over previous
import functools

import jax
import jax.numpy as jnp
from jax import lax
from jax.experimental import pallas as pl
from jax.experimental.pallas import tpu as pltpu
from jax.experimental.pallas import tpu_sc as plsc

_N_NODES = 10000
_D = 128
_G = 100
_CHUNK = 104
_NCHW = 3
_PER_W = _CHUNK * _NCHW
_NW = 32
_TAIL = _N_NODES - _PER_W * _NW
_TAIL_BASE = _PER_W * _NW
_BROWS = 79
_BPAD = _BROWS * 128 - _N_NODES


def _sc_segment_sums(x, x_reactant, batch_i32, z128):
    mesh = plsc.VectorSubcoreMesh(core_axis_name="c", subcore_axis_name="s")

    @functools.partial(
        pl.kernel,
        mesh=mesh,
        out_type=(
            jax.ShapeDtypeStruct((2, _G, _D), jnp.float32),
            jax.ShapeDtypeStruct((2, _G, _D), jnp.float32),
        ),
        scratch_types=[
            pltpu.VMEM((_CHUNK, _D), jnp.float32),
            pltpu.VMEM((_CHUNK, _D), jnp.float32),
            pltpu.VMEM((_CHUNK, _D), jnp.float32),
            pltpu.VMEM((_CHUNK, _D), jnp.float32),
            pltpu.VMEM((_CHUNK, _D), jnp.float32),
            pltpu.VMEM((_CHUNK, _D), jnp.float32),
            pltpu.VMEM((_CHUNK,), jnp.int32),
            pltpu.VMEM((_CHUNK,), jnp.int32),
            pltpu.VMEM((_CHUNK,), jnp.int32),
            pltpu.VMEM((_TAIL, _D), jnp.float32),
            pltpu.VMEM((_TAIL, _D), jnp.float32),
            pltpu.VMEM((_TAIL,), jnp.int32),
            pltpu.VMEM_SHARED((_G, _D), jnp.float32),
            pltpu.VMEM_SHARED((_G, _D), jnp.float32),
            pltpu.SemaphoreType.DMA,
            pltpu.SemaphoreType.DMA,
            pltpu.SemaphoreType.DMA,
            pltpu.SemaphoreType.DMA,
        ],
    )
    def seg_sum_kernel(x_hbm, xr_hbm, b_hbm, z128_hbm,
                       outx_hbm, outxr_hbm,
                       xa0, xa1, xa2, xb0, xb1, xb2, i0, i1, i2,
                       xt, xrt, it,
                       accx_s, accxr_s,
                       sg0, sg1, sg2, ss):
        c = lax.axis_index("c")
        s = lax.axis_index("s")
        wid = s * 2 + c
        xa = (xa0, xa1, xa2)
        xb = (xb0, xb1, xb2)
        ii = (i0, i1, i2)
        sg = (sg0, sg1, sg2)

        @pl.when(s == 0)
        def _zero_shared():
            pltpu.sync_copy(z128_hbm, accx_s)
            pltpu.sync_copy(z128_hbm, accxr_s)

        base0 = wid * _PER_W
        gathers = []
        for t in range(_NCHW):
            b = pl.multiple_of(base0 + t * _CHUNK, 8)
            gathers.append((
                pltpu.async_copy(x_hbm.at[pl.ds(b, _CHUNK), :], xa[t], sg[t]),
                pltpu.async_copy(xr_hbm.at[pl.ds(b, _CHUNK), :], xb[t], sg[t]),
                pltpu.async_copy(b_hbm.at[pl.ds(b, _CHUNK)], ii[t], sg[t]),
            ))

        plsc.subcore_barrier()

        scatters = []
        for t in range(_NCHW):
            for d in gathers[t]:
                d.wait()
            scatters.append(
                pltpu.async_copy(xa[t], accx_s.at[ii[t]], ss, add=True))
            scatters.append(
                pltpu.async_copy(xb[t], accxr_s.at[ii[t]], ss, add=True))
        for d in scatters:
            d.wait()

        @pl.when(wid == _NW - 1)
        def _tail():
            pltpu.sync_copy(x_hbm.at[pl.ds(_TAIL_BASE, _TAIL), :], xt)
            pltpu.sync_copy(xr_hbm.at[pl.ds(_TAIL_BASE, _TAIL), :], xrt)
            pltpu.sync_copy(b_hbm.at[pl.ds(_TAIL_BASE, _TAIL)], it)
            pltpu.sync_copy(xt, accx_s.at[it], add=True)
            pltpu.sync_copy(xrt, accxr_s.at[it], add=True)

        plsc.subcore_barrier()

        @pl.when(s == 0)
        def _write_out():
            pltpu.sync_copy(accx_s, outx_hbm.at[c])
            pltpu.sync_copy(accxr_s, outxr_hbm.at[c])

    return seg_sum_kernel(x, x_reactant, batch_i32, z128)


def _finalize(u, u_reactant, accx, accxr, bpad):
    def body(u_ref, ur_ref, ax_ref, axr_ref, b_ref, o_ref):
        ssum = (ax_ref[0] - axr_ref[0]) + (ax_ref[1] - axr_ref[1])
        gcol = lax.broadcasted_iota(jnp.int32, (_G, 1), 0)
        counts = jnp.zeros((_G, 128), jnp.float32)
        for r in range(_BROWS):
            row = b_ref[r, :].reshape(1, 128)
            counts = counts + (row == gcol).astype(jnp.float32)
        denom = jnp.maximum(jnp.sum(counts, axis=1, keepdims=True), 1.0)
        du = u_ref[...] - ur_ref[...]
        o_ref[...] = jnp.concatenate(
            [du, ssum / denom, jnp.zeros((_G, 16), jnp.float32)], axis=1)

    return pl.pallas_call(
        body,
        out_shape=jax.ShapeDtypeStruct((_G, 32 + _D + 16), jnp.float32),
    )(u, u_reactant, accx, accxr, bpad)


def kernel(x, x_reactant, edge_index, edge_attr, u, u_reactant, batch):
    del edge_index, edge_attr
    batch_i32 = batch.astype(jnp.int32)
    z128 = jnp.zeros((_G, _D), jnp.float32)
    accx, accxr = _sc_segment_sums(x, x_reactant, batch_i32, z128)
    bpad = jnp.pad(batch_i32, (0, _BPAD),
                   constant_values=127).reshape(_BROWS, 128)
    return _finalize(u, u_reactant, accx, accxr, bpad)


# speedup vs baseline: 88.9431x; 1.0634x over previous
import functools

import jax
import jax.numpy as jnp
from jax import lax
from jax.experimental import pallas as pl
from jax.experimental.pallas import tpu as pltpu
from jax.experimental.pallas import tpu_sc as plsc

_N_NODES = 10000
_D = 128
_G = 100
_CHUNK = 104
_NCHW = 3
_PER_W = _CHUNK * _NCHW
_NW = 32
_TAIL = _N_NODES - _PER_W * _NW
_TAIL_BASE = _PER_W * _NW
_BROWS = 79
_BPAD = _BROWS * 128 - _N_NODES


def _sub_rows(dst, src, nrows):
    def body(r, carry):
        for k in range(_D // 16):
            sl = pl.ds(k * 16, 16)
            dst[r, sl] = dst[r, sl] - src[r, sl]
        return carry

    lax.fori_loop(0, nrows, body, 0)


def _sc_segment_sums(x, x_reactant, batch_i32, z128):
    mesh = plsc.VectorSubcoreMesh(core_axis_name="c", subcore_axis_name="s")

    @functools.partial(
        pl.kernel,
        mesh=mesh,
        out_type=jax.ShapeDtypeStruct((2, _G, _D), jnp.float32),
        scratch_types=[
            pltpu.VMEM((_CHUNK, _D), jnp.float32),
            pltpu.VMEM((_CHUNK, _D), jnp.float32),
            pltpu.VMEM((_CHUNK, _D), jnp.float32),
            pltpu.VMEM((_CHUNK, _D), jnp.float32),
            pltpu.VMEM((_CHUNK, _D), jnp.float32),
            pltpu.VMEM((_CHUNK, _D), jnp.float32),
            pltpu.VMEM((_CHUNK,), jnp.int32),
            pltpu.VMEM((_CHUNK,), jnp.int32),
            pltpu.VMEM((_CHUNK,), jnp.int32),
            pltpu.VMEM((_TAIL, _D), jnp.float32),
            pltpu.VMEM((_TAIL, _D), jnp.float32),
            pltpu.VMEM((_TAIL,), jnp.int32),
            pltpu.VMEM_SHARED((_G, _D), jnp.float32),
            pltpu.SemaphoreType.DMA,
            pltpu.SemaphoreType.DMA,
            pltpu.SemaphoreType.DMA,
            pltpu.SemaphoreType.DMA,
        ],
    )
    def seg_sum_kernel(x_hbm, xr_hbm, b_hbm, z128_hbm,
                       outd_hbm,
                       xa0, xa1, xa2, xb0, xb1, xb2, i0, i1, i2,
                       xt, xrt, it,
                       accd_s,
                       sg0, sg1, sg2, ss):
        c = lax.axis_index("c")
        s = lax.axis_index("s")
        wid = s * 2 + c
        xa = (xa0, xa1, xa2)
        xb = (xb0, xb1, xb2)
        ii = (i0, i1, i2)
        sg = (sg0, sg1, sg2)

        @pl.when(s == 0)
        def _zero_shared():
            pltpu.sync_copy(z128_hbm, accd_s)

        base0 = wid * _PER_W
        gathers = []
        for t in range(_NCHW):
            b = pl.multiple_of(base0 + t * _CHUNK, 8)
            gathers.append((
                pltpu.async_copy(x_hbm.at[pl.ds(b, _CHUNK), :], xa[t], sg[t]),
                pltpu.async_copy(xr_hbm.at[pl.ds(b, _CHUNK), :], xb[t], sg[t]),
                pltpu.async_copy(b_hbm.at[pl.ds(b, _CHUNK)], ii[t], sg[t]),
            ))

        plsc.subcore_barrier()

        scatters = []
        for t in range(_NCHW):
            for d in gathers[t]:
                d.wait()
            _sub_rows(xa[t], xb[t], _CHUNK)
            scatters.append(
                pltpu.async_copy(xa[t], accd_s.at[ii[t]], ss, add=True))
        for d in scatters:
            d.wait()

        @pl.when(wid == _NW - 1)
        def _tail():
            pltpu.sync_copy(x_hbm.at[pl.ds(_TAIL_BASE, _TAIL), :], xt)
            pltpu.sync_copy(xr_hbm.at[pl.ds(_TAIL_BASE, _TAIL), :], xrt)
            pltpu.sync_copy(b_hbm.at[pl.ds(_TAIL_BASE, _TAIL)], it)
            _sub_rows(xt, xrt, _TAIL)
            pltpu.sync_copy(xt, accd_s.at[it], add=True)

        plsc.subcore_barrier()

        @pl.when(s == 0)
        def _write_out():
            pltpu.sync_copy(accd_s, outd_hbm.at[c])

    return seg_sum_kernel(x, x_reactant, batch_i32, z128)


def _finalize(u, u_reactant, accd, bpad):
    def body(u_ref, ur_ref, ad_ref, b_ref, o_ref):
        ssum = ad_ref[0] + ad_ref[1]
        gcol = lax.broadcasted_iota(jnp.int32, (_G, 1), 0)
        counts = jnp.zeros((_G, 128), jnp.float32)
        for r in range(_BROWS):
            row = b_ref[r, :].reshape(1, 128)
            counts = counts + (row == gcol).astype(jnp.float32)
        denom = jnp.maximum(jnp.sum(counts, axis=1, keepdims=True), 1.0)
        du = u_ref[...] - ur_ref[...]
        o_ref[...] = jnp.concatenate(
            [du, ssum / denom, jnp.zeros((_G, 16), jnp.float32)], axis=1)

    return pl.pallas_call(
        body,
        out_shape=jax.ShapeDtypeStruct((_G, 32 + _D + 16), jnp.float32),
    )(u, u_reactant, accd, bpad)


def kernel(x, x_reactant, edge_index, edge_attr, u, u_reactant, batch):
    del edge_index, edge_attr
    batch_i32 = batch.astype(jnp.int32)
    z128 = jnp.zeros((_G, _D), jnp.float32)
    accd = _sc_segment_sums(x, x_reactant, batch_i32, z128)
    bpad = jnp.pad(batch_i32, (0, _BPAD),
                   constant_values=127).reshape(_BROWS, 128)
    return _finalize(u, u_reactant, accd, bpad)


# speedup vs baseline: 90.1440x; 1.0135x over previous
import functools

import jax
import jax.numpy as jnp
from jax import lax
from jax.experimental import pallas as pl
from jax.experimental.pallas import tpu as pltpu
from jax.experimental.pallas import tpu_sc as plsc

_N_NODES = 10000
_D = 128
_G = 100
_CHUNK = 104
_NCHW = 3
_PER_W = _CHUNK * _NCHW
_NW = 32
_TAIL = _N_NODES - _PER_W * _NW
_TAIL_BASE = _PER_W * _NW
_BROWS = 79
_BPAD = _BROWS * 128 - _N_NODES


def _sub_rows(dst, src, nrows):
    def body(r, carry):
        for k in range(_D // 16):
            sl = pl.ds(k * 16, 16)
            dst[r, sl] = dst[r, sl] - src[r, sl]
        return carry

    lax.fori_loop(0, nrows, body, 0)


def _sc_segment_sums(x, x_reactant, batch_i32, z128):
    mesh = plsc.VectorSubcoreMesh(core_axis_name="c", subcore_axis_name="s")

    @functools.partial(
        pl.kernel,
        mesh=mesh,
        out_type=jax.ShapeDtypeStruct((2, _G, _D), jnp.float32),
        scratch_types=[
            pltpu.VMEM((_CHUNK, _D), jnp.float32),
            pltpu.VMEM((_CHUNK, _D), jnp.float32),
            pltpu.VMEM((_CHUNK, _D), jnp.float32),
            pltpu.VMEM((_CHUNK, _D), jnp.float32),
            pltpu.VMEM((_CHUNK, _D), jnp.float32),
            pltpu.VMEM((_CHUNK, _D), jnp.float32),
            pltpu.VMEM((_CHUNK,), jnp.int32),
            pltpu.VMEM((_CHUNK,), jnp.int32),
            pltpu.VMEM((_CHUNK,), jnp.int32),
            pltpu.VMEM((_TAIL, _D), jnp.float32),
            pltpu.VMEM((_TAIL, _D), jnp.float32),
            pltpu.VMEM((_TAIL,), jnp.int32),
            pltpu.VMEM_SHARED((_G, _D), jnp.float32),
            pltpu.SemaphoreType.DMA,
            pltpu.SemaphoreType.DMA,
            pltpu.SemaphoreType.DMA,
            pltpu.SemaphoreType.DMA,
        ],
    )
    def seg_sum_kernel(x_hbm, xr_hbm, b_hbm, z128_hbm,
                       outd_hbm,
                       xa0, xa1, xa2, xb0, xb1, xb2, i0, i1, i2,
                       xt, xrt, it,
                       accd_s,
                       sg0, sg1, sg2, ss):
        c = lax.axis_index("c")
        s = lax.axis_index("s")
        wid = s * 2 + c
        xa = (xa0, xa1, xa2)
        xb = (xb0, xb1, xb2)
        ii = (i0, i1, i2)
        sg = (sg0, sg1, sg2)

        @pl.when(s == 0)
        def _zero_shared():
            pltpu.sync_copy(z128_hbm, accd_s)

        base0 = wid * _PER_W
        gathers = []
        for t in range(_NCHW):
            b = pl.multiple_of(base0 + t * _CHUNK, 8)
            gathers.append((
                pltpu.async_copy(x_hbm.at[pl.ds(b, _CHUNK), :], xa[t], sg[t]),
                pltpu.async_copy(xr_hbm.at[pl.ds(b, _CHUNK), :], xb[t], sg[t]),
                pltpu.async_copy(b_hbm.at[pl.ds(b, _CHUNK)], ii[t], sg[t]),
            ))

        plsc.subcore_barrier()

        scatters = []
        for t in range(_NCHW):
            for d in gathers[t]:
                d.wait()
            _sub_rows(xa[t], xb[t], _CHUNK)
            scatters.append(
                pltpu.async_copy(xa[t], accd_s.at[ii[t]], ss, add=True))
        for d in scatters:
            d.wait()

        @pl.when(wid == _NW - 1)
        def _tail():
            pltpu.sync_copy(x_hbm.at[pl.ds(_TAIL_BASE, _TAIL), :], xt)
            pltpu.sync_copy(xr_hbm.at[pl.ds(_TAIL_BASE, _TAIL), :], xrt)
            pltpu.sync_copy(b_hbm.at[pl.ds(_TAIL_BASE, _TAIL)], it)
            _sub_rows(xt, xrt, _TAIL)
            pltpu.sync_copy(xt, accd_s.at[it], add=True)

        plsc.subcore_barrier()

        @pl.when(s == 0)
        def _write_out():
            pltpu.sync_copy(accd_s, outd_hbm.at[c])

    return seg_sum_kernel(x, x_reactant, batch_i32, z128)


def _count_denom(bpad):
    def body(b_ref, o_ref):
        gcol = lax.broadcasted_iota(jnp.int32, (_G, 1), 0)
        counts = jnp.zeros((_G, 128), jnp.float32)
        for r in range(_BROWS):
            row = b_ref[r, :].reshape(1, 128)
            counts = counts + (row == gcol).astype(jnp.float32)
        total = jnp.sum(counts, axis=1, keepdims=True)
        o_ref[...] = jnp.maximum(total, 1.0) * jnp.ones((_G, _D), jnp.float32)

    return pl.pallas_call(
        body,
        out_shape=jax.ShapeDtypeStruct((_G, _D), jnp.float32),
    )(bpad)


def _finalize(u, u_reactant, accd, denom):
    def body(u_ref, ur_ref, ad_ref, dn_ref, o_ref):
        ssum = ad_ref[0] + ad_ref[1]
        du = u_ref[...] - ur_ref[...]
        o_ref[...] = jnp.concatenate(
            [du, ssum / dn_ref[...], jnp.zeros((_G, 16), jnp.float32)],
            axis=1)

    return pl.pallas_call(
        body,
        out_shape=jax.ShapeDtypeStruct((_G, 32 + _D + 16), jnp.float32),
    )(u, u_reactant, accd, denom)


def kernel(x, x_reactant, edge_index, edge_attr, u, u_reactant, batch):
    del edge_index, edge_attr
    batch_i32 = batch.astype(jnp.int32)
    z128 = jnp.zeros((_G, _D), jnp.float32)
    accd = _sc_segment_sums(x, x_reactant, batch_i32, z128)
    bpad = jnp.pad(batch_i32, (0, _BPAD),
                   constant_values=127).reshape(_BROWS, 128)
    denom = _count_denom(bpad)
    return _finalize(u, u_reactant, accd, denom)
